# Initial kernel scaffold; baseline (speedup 1.0000x reference)
import jax
import jax.numpy as jnp
from jax.experimental import pallas as pl


def kernel(inputs, targets):
    raise NotImplementedError("write your pallas kernel here")


# speedup vs baseline: 1.4480x; 1.4480x over previous
import jax
import jax.numpy as jnp
from jax.experimental import pallas as pl
from jax.experimental.pallas import tpu as pltpu

_B, _N, _L = 16, 1024, 8
_CH = 256


def _row_body(vals_ref, labs_ref, out_ref):
    b = pl.program_id(0)
    vals = vals_ref[0, 0, :]
    labs = labs_ref[0, 0, :]

    counts = []
    for c in range(_N // _CH):
        vi = vals[c * _CH:(c + 1) * _CH][:, None]
        li = labs[c * _CH:(c + 1) * _CH][:, None]
        eq = (vi == vals[None, :]) & (li == labs[None, :])
        counts.append(jnp.sum(eq.astype(jnp.float32), axis=1))
    count = jnp.concatenate(counts)

    total = jnp.float32(0.0)
    for g in range(_L):
        m = labs == jnp.float32(g)
        gsize = jnp.sum(jnp.where(m, 1.0, 0.0))
        gsum = jnp.sum(jnp.where(m, vals, 0.0))
        gmax = jnp.max(jnp.where(m, count, -1.0))
        mode = jnp.min(jnp.where(m & (count == gmax), vals, jnp.inf))
        total = total + jnp.where(gsize > 0, gsum - gsize * mode, 0.0)

    @pl.when(b == 0)
    def _():
        out_ref[0, 0] = jnp.float32(0.0)

    out_ref[0, 0] += total / jnp.float32(_B * _N)


def kernel(inputs, targets):
    out = pl.pallas_call(
        _row_body,
        grid=(_B,),
        in_specs=[
            pl.BlockSpec((1, 1, _N), lambda b: (b, 0, 0)),
            pl.BlockSpec((1, 1, _N), lambda b: (b, 0, 0)),
        ],
        out_specs=pl.BlockSpec((1, 1), lambda b: (0, 0), memory_space=pltpu.SMEM),
        out_shape=jax.ShapeDtypeStruct((1, 1), jnp.float32),
    )(inputs.reshape(_B, 1, _N), targets.reshape(_B, 1, _N))
    return out[0, 0]
